# baseline (speedup 1.0000x reference)
import functools

import jax
import jax.numpy as jnp
from jax import lax
from jax.experimental import pallas as pl
from jax.experimental.pallas import tpu as pltpu
from jax.experimental.pallas import tpu_sc as plsc

_NC, _NS = 2, 16
_NW = _NC * _NS
_NBUF = 3


def _sc_gather_build(B, S, D):
    bpw = B // _NW
    s_pad = S + 8
    mesh = plsc.VectorSubcoreMesh(core_axis_name="c", subcore_axis_name="s")

    @functools.partial(
        pl.kernel,
        mesh=mesh,
        compiler_params=pltpu.CompilerParams(use_tc_tiling_on_sc=False),
        out_type=(
            jax.ShapeDtypeStruct((B, S, D), jnp.float32),
            jax.ShapeDtypeStruct((B, S + 1, D), jnp.float32),
        ),
        scratch_types=[
            pltpu.VMEM((bpw * S,), jnp.int32),
            pltpu.VMEM((bpw * s_pad,), jnp.int32),
        ]
        + [pltpu.VMEM((s_pad, D), jnp.float32) for _ in range(2 * _NBUF)]
        + [pltpu.SemaphoreType.DMA for _ in range(2 * _NBUF)],
    )
    def sc_kernel(cat_idx, txt_idx, wcat, wnlp, out_cat, out_txt,
                  ic_all, it_all, rc0, rc1, rc2, rt0, rt1, rt2,
                  g0, g1, g2, s0, s1, s2):
        wid = lax.axis_index("s") * _NC + lax.axis_index("c")
        base_b = wid * bpw
        pltpu.sync_copy(cat_idx.at[pl.ds(base_b * S, bpw * S)], ic_all)
        pltpu.sync_copy(txt_idx.at[pl.ds(base_b * s_pad, bpw * s_pad)],
                        it_all)

        def run_batches(table, idx_all, out3d, stride, subs, store_n, rows,
                        gsems, ssems):
            def cp_gathers(i, s):
                return [
                    pltpu.make_async_copy(
                        table.at[idx_all.at[pl.ds(i * stride + off, sz)]],
                        rows[s].at[pl.ds(off, sz)], gsems[s])
                    for off, sz in subs
                ]

            def cp_store(i, s):
                return pltpu.make_async_copy(
                    rows[s].at[pl.ds(0, store_n)],
                    out3d.at[base_b + i], ssems[s])

            def step(i, s, issue_next):
                for c in cp_gathers(i, s):
                    c.wait()
                st = cp_store(i, s)
                st.start()
                st.wait()
                if issue_next:
                    for c in cp_gathers(i + _NBUF, s):
                        c.start()

            for k in range(_NBUF):
                for c in cp_gathers(k, k):
                    c.start()

            n_main = max((bpw - _NBUF) // _NBUF, 0) * _NBUF

            def body(g, carry):
                for j in range(_NBUF):
                    step(g * _NBUF + j, j, True)
                return carry

            lax.fori_loop(0, n_main // _NBUF, body, 0)
            for i in range(n_main, bpw):
                step(i, i % _NBUF, i + _NBUF < bpw)

        run_batches(wcat, ic_all, out_cat, S, ((0, 104), (104, 96)), S,
                    (rc0, rc1, rc2), (g0, g1, g2), (s0, s1, s2))
        run_batches(wnlp, it_all, out_txt, s_pad, ((0, 104), (104, 104)),
                    S + 1, (rt0, rt1, rt2), (g0, g1, g2), (s0, s1, s2))

    return sc_kernel


def _dense_body(t_ref, w_ref, b_ref, g_ref, og_ref, ot_ref):
    og_ref[...] = jnp.broadcast_to(g_ref[...], og_ref.shape)
    ot_ref[...] = t_ref[...] * w_ref[...] + b_ref[...]


def kernel(target, cat_feat, text, global_token, W_lin, b_lin, W_cat, W_nlp):
    B, S, _ = target.shape
    D = global_token.shape[-1]
    s_pad = S + 8

    cat_idx = cat_feat.reshape(B * S).astype(jnp.int32)
    gt2 = global_token.reshape(1, D).astype(jnp.float32)
    wnlp_ext = jnp.concatenate([W_nlp.astype(jnp.float32), gt2], axis=0)
    gt_col = jnp.full((B, 1), W_nlp.shape[0], dtype=jnp.int32)
    pad_cols = jnp.zeros((B, s_pad - S - 1), dtype=jnp.int32)
    txt_idx = jnp.concatenate(
        [gt_col, text.astype(jnp.int32), pad_cols], axis=1).reshape(-1)

    out_cat, out_txt = _sc_gather_build(B, S, D)(
        cat_idx, txt_idx, W_cat, wnlp_ext)

    t3d = jnp.transpose(target, (1, 2, 0))
    w3d = W_lin.reshape(1, D, 1)
    b3d = b_lin.reshape(1, D, 1)
    g3d = global_token.reshape(1, D, 1)

    SB = 8
    og_p, ot_p = pl.pallas_call(
        _dense_body,
        grid=(S // SB,),
        in_specs=[
            pl.BlockSpec((SB, 1, B), lambda i: (i, 0, 0)),
            pl.BlockSpec((1, D, 1), lambda i: (0, 0, 0)),
            pl.BlockSpec((1, D, 1), lambda i: (0, 0, 0)),
            pl.BlockSpec((1, D, 1), lambda i: (0, 0, 0)),
        ],
        out_specs=[
            pl.BlockSpec((SB, D, B), lambda i: (i, 0, 0)),
            pl.BlockSpec((SB, D, B), lambda i: (i, 0, 0)),
        ],
        out_shape=[
            jax.ShapeDtypeStruct((S, D, B), jnp.float32),
            jax.ShapeDtypeStruct((S, D, B), jnp.float32),
        ],
    )(t3d, w3d, b3d, g3d)

    out_global = jnp.transpose(og_p, (2, 0, 1))
    out_target = jnp.transpose(ot_p, (2, 0, 1))
    return (out_global, out_target, out_cat, out_txt)


# speedup vs baseline: 1.3460x; 1.3460x over previous
import functools

import jax
import jax.numpy as jnp
from jax import lax
from jax.experimental import pallas as pl
from jax.experimental.pallas import tpu as pltpu
from jax.experimental.pallas import tpu_sc as plsc

_NC, _NS = 2, 16
_NW = _NC * _NS
_NBUF = 3


def _sc_gather_one(B, S_out, D, stride, subs):
    bpw = B // _NW

    mesh = plsc.VectorSubcoreMesh(core_axis_name="c", subcore_axis_name="s")

    @functools.partial(
        pl.kernel,
        mesh=mesh,
        compiler_params=pltpu.CompilerParams(use_tc_tiling_on_sc=False),
        out_type=jax.ShapeDtypeStruct((B, S_out, D), jnp.float32),
        scratch_types=[pltpu.VMEM((bpw * stride,), jnp.int32)]
        + [pltpu.VMEM((stride, D), jnp.float32) for _ in range(_NBUF)]
        + [pltpu.SemaphoreType.DMA for _ in range(2 * _NBUF)],
    )
    def sc_kernel(idx, table, out3d, i_all, r0, r1, r2, g0, g1, g2,
                  s0, s1, s2):
        rows = (r0, r1, r2)
        gsems = (g0, g1, g2)
        ssems = (s0, s1, s2)
        wid = lax.axis_index("s") * _NC + lax.axis_index("c")
        base_b = wid * bpw
        pltpu.sync_copy(idx.at[pl.ds(base_b * stride, bpw * stride)], i_all)

        def cp_gathers(i, s):
            return [
                pltpu.make_async_copy(
                    table.at[i_all.at[pl.ds(i * stride + off, sz)]],
                    rows[s].at[pl.ds(off, sz)], gsems[s])
                for off, sz in subs
            ]

        def cp_store(i, s):
            return pltpu.make_async_copy(
                rows[s].at[pl.ds(0, S_out)], out3d.at[base_b + i], ssems[s])

        def step(i, s, issue_next):
            for c in cp_gathers(i, s):
                c.wait()
            st = cp_store(i, s)
            st.start()
            st.wait()
            if issue_next:
                for c in cp_gathers(i + _NBUF, s):
                    c.start()

        for k in range(_NBUF):
            for c in cp_gathers(k, k):
                c.start()

        n_main = max((bpw - _NBUF) // _NBUF, 0) * _NBUF

        def body(g, carry):
            for j in range(_NBUF):
                step(g * _NBUF + j, j, True)
            return carry

        lax.fori_loop(0, n_main // _NBUF, body, 0)
        for i in range(n_main, bpw):
            step(i, i % _NBUF, i + _NBUF < bpw)

    return sc_kernel


def _transpose_body(lo_ref, hi_ref, out_ref):
    out_ref[:, 0:64] = lo_ref[...].T
    out_ref[:, 64:128] = hi_ref[...].T


def _dense_body(t_ref, w_ref, b_ref, g_ref, og_ref, ot_ref):
    og_ref[...] = jnp.broadcast_to(g_ref[...], og_ref.shape)
    ot_ref[...] = t_ref[...] * w_ref[...] + b_ref[...]


def kernel(target, cat_feat, text, global_token, W_lin, b_lin, W_cat, W_nlp):
    B, S, _ = target.shape
    D = global_token.shape[-1]
    V = W_cat.shape[0]
    s_pad = S + 8

    V2P = 501760
    ci = cat_feat.reshape(B * S).astype(jnp.int32)
    cat_idx = jnp.where(ci < V2P, 2 * ci, 2 * ci - (2 * V2P - 1))
    gt2 = global_token.reshape(1, D).astype(jnp.float32)
    wnlp_ext = jnp.concatenate([W_nlp.astype(jnp.float32), gt2], axis=0)
    gt_col = jnp.full((B, 1), W_nlp.shape[0], dtype=jnp.int32)
    pad_cols = jnp.zeros((B, s_pad - S - 1), dtype=jnp.int32)
    txt_idx = jnp.concatenate(
        [gt_col, text.astype(jnp.int32), pad_cols], axis=1).reshape(-1)

    BKC = 2048
    wT = jnp.transpose(W_cat, (1, 0))
    n_blk = V2P // BKC
    wcat_packed = pl.pallas_call(
        _transpose_body,
        grid=(n_blk,),
        in_specs=[
            pl.BlockSpec((D, BKC), lambda i: (0, i)),
            pl.BlockSpec(
                (D, BKC),
                lambda i, n=n_blk, m=V // BKC: (0, jnp.minimum(n + i, m))),
        ],
        out_specs=pl.BlockSpec((BKC, 2 * D), lambda i: (i, 0)),
        out_shape=jax.ShapeDtypeStruct((V2P, 2 * D), jnp.float32),
    )(wT, wT)
    wcat_compact = wcat_packed.reshape(2 * V2P, D)

    out_txt = _sc_gather_one(B, S + 1, D, s_pad, ((0, 104), (104, 104)))(
        txt_idx, wnlp_ext)
    out_cat = _sc_gather_one(B, S, D, S, ((0, 104), (104, 96)))(
        cat_idx, wcat_compact)

    t3d = jnp.transpose(target, (1, 2, 0))
    w3d = W_lin.reshape(1, D, 1)
    b3d = b_lin.reshape(1, D, 1)
    g3d = global_token.reshape(1, D, 1)

    SB = 8
    og_p, ot_p = pl.pallas_call(
        _dense_body,
        grid=(S // SB,),
        in_specs=[
            pl.BlockSpec((SB, 1, B), lambda i: (i, 0, 0)),
            pl.BlockSpec((1, D, 1), lambda i: (0, 0, 0)),
            pl.BlockSpec((1, D, 1), lambda i: (0, 0, 0)),
            pl.BlockSpec((1, D, 1), lambda i: (0, 0, 0)),
        ],
        out_specs=[
            pl.BlockSpec((SB, D, B), lambda i: (i, 0, 0)),
            pl.BlockSpec((SB, D, B), lambda i: (i, 0, 0)),
        ],
        out_shape=[
            jax.ShapeDtypeStruct((S, D, B), jnp.float32),
            jax.ShapeDtypeStruct((S, D, B), jnp.float32),
        ],
    )(t3d, w3d, b3d, g3d)

    out_global = jnp.transpose(og_p, (2, 0, 1))
    out_target = jnp.transpose(ot_p, (2, 0, 1))
    return (out_global, out_target, out_cat, out_txt)
